# baseline (speedup 1.0000x reference)
import functools

import jax
import jax.numpy as jnp
from jax import lax
from jax.experimental import pallas as pl
from jax.experimental.pallas import tpu as pltpu
from jax.experimental.pallas import tpu_sc as plsc

N = 10000
D = 128
E = 320000

K = 128
NC = 2
NS = 16
NW = NC * NS
CHUNKS = 80
EPT = CHUNKS * K
E_PAD = NW * EPT
NWC = NW * CHUNKS

N_FULL_OUT = N // K
TAIL_ROWS = N - N_FULL_OUT * K


def _sc_aggregate(x, srcs, dsts, ew):
    mesh = plsc.VectorSubcoreMesh(core_axis_name="c", subcore_axis_name="s")

    @functools.partial(
        pl.kernel,
        out_type=jax.ShapeDtypeStruct((NC, N, D), jnp.float32),
        mesh=mesh,
        scratch_types=[
            pltpu.VMEM((K,), jnp.int32),
            pltpu.VMEM((K,), jnp.int32),
            pltpu.VMEM((K,), jnp.int32),
            pltpu.VMEM((K,), jnp.int32),
            pltpu.VMEM((K,), jnp.float32),
            pltpu.VMEM((K,), jnp.float32),
            pltpu.VMEM((K, D), jnp.float32),
            pltpu.VMEM((K, D), jnp.float32),
            pltpu.VMEM_SHARED((N, D), jnp.float32),
            pltpu.SemaphoreType.DMA,
            pltpu.SemaphoreType.DMA,
            pltpu.SemaphoreType.DMA,
            pltpu.SemaphoreType.DMA,
        ],
    )
    def agg(x_hbm, src_hbm, dst_hbm, ew_hbm, part_hbm,
            sb0, sb1, db0, db1, ewb0, ewb1, rows0, rows1, acc_sh,
            isem0, isem1, gsem0, gsem1):
        cid = lax.axis_index("c")
        sid = lax.axis_index("s")
        wid = cid * NS + sid
        base_chunk = wid * CHUNKS

        sbs = (sb0, sb1)
        dbs = (db0, db1)
        ewbs = (ewb0, ewb1)
        rowss = (rows0, rows1)
        isems = (isem0, isem1)
        gsems = (gsem0, gsem1)

        zero16 = jnp.zeros((16,), jnp.float32)

        @pl.loop(0, K)
        def _zero_rows(r):
            for c in range(D // 16):
                rows0[r, pl.ds(c * 16, 16)] = zero16

        @pl.loop(sid, N_FULL_OUT, step=NS)
        def _zero_acc(t):
            pltpu.sync_copy(rows0, acc_sh.at[pl.ds(t * K, K)])

        @pl.when(sid == 0)
        def _zero_tail():
            pltpu.sync_copy(rows0.at[pl.ds(0, TAIL_ROWS)],
                            acc_sh.at[pl.ds(N_FULL_OUT * K, TAIL_ROWS)])

        plsc.subcore_barrier()

        def scale_rows(rows, ewb):
            @pl.loop(0, K // 16)
            def _scale_group(g):
                ewg = ewb[pl.ds(g * 16, 16)]
                for jj in range(16):
                    e = g * 16 + jj
                    w = ewg[jj]
                    for c in range(D // 16):
                        rows[e, pl.ds(c * 16, 16)] = (
                            rows[e, pl.ds(c * 16, 16)] * w)

        pltpu.sync_copy(src_hbm.at[pl.ds(base_chunk * K, K)], sb0)
        pltpu.sync_copy(dst_hbm.at[pl.ds(base_chunk * K, K)], db0)
        pltpu.sync_copy(ew_hbm.at[pl.ds(base_chunk * K, K)], ewb0)
        pltpu.async_copy(x_hbm.at[sb0], rows0, gsem0)

        def halfstep(j, cur):
            nxt = 1 - cur
            nbase = (base_chunk + j + 1) * K
            pltpu.make_async_copy(
                x_hbm.at[sbs[cur]], rowss[cur], gsems[cur]).wait()
            pltpu.async_copy(src_hbm.at[pl.ds(nbase, K)], sbs[nxt], isems[nxt])
            pltpu.async_copy(dst_hbm.at[pl.ds(nbase, K)], dbs[nxt], isems[nxt])
            pltpu.async_copy(ew_hbm.at[pl.ds(nbase, K)], ewbs[nxt], isems[nxt])
            scale_rows(rowss[cur], ewbs[cur])
            pltpu.make_async_copy(src_hbm.at[pl.ds(nbase, K)], sbs[nxt],
                                  isems[nxt]).wait()
            pltpu.make_async_copy(dst_hbm.at[pl.ds(nbase, K)], dbs[nxt],
                                  isems[nxt]).wait()
            pltpu.make_async_copy(ew_hbm.at[pl.ds(nbase, K)], ewbs[nxt],
                                  isems[nxt]).wait()
            pltpu.async_copy(x_hbm.at[sbs[nxt]], rowss[nxt], gsems[nxt])
            pltpu.sync_copy(rowss[cur], acc_sh.at[dbs[cur]], add=True)

        @pl.loop(0, CHUNKS // 2)
        def _pair(t):
            halfstep(2 * t, 0)
            halfstep(2 * t + 1, 1)

        pltpu.make_async_copy(x_hbm.at[sb0], rows0, gsem0).wait()

        plsc.subcore_barrier()

        @pl.loop(sid, N_FULL_OUT, step=NS)
        def _copy_out(t):
            row0 = t * K
            pltpu.sync_copy(acc_sh.at[pl.ds(row0, K)], rows0)
            pltpu.sync_copy(rows0, part_hbm.at[cid, pl.ds(row0, K)])

        @pl.when(sid == 0)
        def _copy_tail():
            row0 = N_FULL_OUT * K
            pltpu.sync_copy(acc_sh.at[pl.ds(row0, TAIL_ROWS)],
                            rows1.at[pl.ds(0, TAIL_ROWS)])
            pltpu.sync_copy(rows1.at[pl.ds(0, TAIL_ROWS)],
                            part_hbm.at[cid, pl.ds(row0, TAIL_ROWS)])

    return agg(x, srcs, dsts, ew)


def _combine_body(p_ref, w_ref, o_ref):
    o_ref[...] = (p_ref[0] + p_ref[1]) * w_ref[...]


def _tc_combine(part, W):
    blk = 2000
    return pl.pallas_call(
        _combine_body,
        out_shape=jax.ShapeDtypeStruct((N, D), jnp.float32),
        grid=(N // blk,),
        in_specs=[
            pl.BlockSpec((NC, blk, D), lambda i: (0, i, 0)),
            pl.BlockSpec((1, D), lambda i: (0, 0)),
        ],
        out_specs=pl.BlockSpec((blk, D), lambda i: (i, 0)),
    )(part, W.reshape(1, D))


def kernel(x, edge_index, edge_weight, W):
    dst = edge_index[0]
    src = edge_index[1]
    pad = E_PAD - E + K
    src_p = jnp.concatenate([src, jnp.zeros((pad,), jnp.int32)])
    dst_p = jnp.concatenate([dst, jnp.zeros((pad,), jnp.int32)])
    ew_p = jnp.concatenate([edge_weight, jnp.zeros((pad,), jnp.float32)])
    part = _sc_aggregate(x, src_p, dst_p, ew_p)
    return _tc_combine(part, W)


# speedup vs baseline: 1.0757x; 1.0757x over previous
import functools

import jax
import jax.numpy as jnp
from jax import lax
from jax.experimental import pallas as pl
from jax.experimental.pallas import tpu as pltpu
from jax.experimental.pallas import tpu_sc as plsc

N = 10000
D = 128
E = 320000

K = 128
NC = 2
NS = 16
NW = NC * NS
CH0 = 104
CH1 = 56
NWC = NS * (CH0 + CH1)
E_PAD = NWC * K

N_FULL_OUT = N // K
TAIL_ROWS = N - N_FULL_OUT * K


def _sc_aggregate(x, srcs, dsts, ew):
    mesh = plsc.VectorSubcoreMesh(core_axis_name="c", subcore_axis_name="s")

    @functools.partial(
        pl.kernel,
        out_type=jax.ShapeDtypeStruct((NC, N, D), jnp.float32),
        mesh=mesh,
        compiler_params=pltpu.CompilerParams(use_tc_tiling_on_sc=False),
        scratch_types=[
            pltpu.VMEM((K,), jnp.int32),
            pltpu.VMEM((K,), jnp.int32),
            pltpu.VMEM((K,), jnp.int32),
            pltpu.VMEM((K,), jnp.int32),
            pltpu.VMEM((K,), jnp.float32),
            pltpu.VMEM((K,), jnp.float32),
            pltpu.VMEM((K, D), jnp.float32),
            pltpu.VMEM((K, D), jnp.float32),
            pltpu.VMEM_SHARED((N, D), jnp.float32),
            pltpu.SemaphoreType.DMA,
            pltpu.SemaphoreType.DMA,
            pltpu.SemaphoreType.DMA,
            pltpu.SemaphoreType.DMA,
            pltpu.SemaphoreType.DMA,
            pltpu.SemaphoreType.DMA,
        ],
    )
    def agg(x_hbm, src_hbm, dst_hbm, ew_hbm, part_hbm,
            sb0, sb1, db0, db1, ewb0, ewb1, rows0, rows1, acc_sh,
            isem0, isem1, gsem0, gsem1, hsem0, hsem1):
        cid = lax.axis_index("c")
        sid = lax.axis_index("s")
        base_chunk = jnp.where(cid == 0, sid * CH0, NS * CH0 + sid * CH1)
        n_pairs = jnp.where(cid == 0, CH0 // 2, CH1 // 2)

        sbs = (sb0, sb1)
        dbs = (db0, db1)
        ewbs = (ewb0, ewb1)
        rowss = (rows0, rows1)
        isems = (isem0, isem1)
        gsems = (gsem0, gsem1)
        hsems = (hsem0, hsem1)

        zero16 = jnp.zeros((16,), jnp.float32)

        @pl.loop(0, K)
        def _zero_rows(r):
            for c in range(D // 16):
                rows0[r, pl.ds(c * 16, 16)] = zero16

        @pl.loop(sid, N_FULL_OUT, step=NS)
        def _zero_acc(t):
            pltpu.sync_copy(rows0, acc_sh.at[pl.ds(t * K, K)])

        @pl.when(sid == 0)
        def _zero_tail():
            pltpu.sync_copy(rows0.at[pl.ds(0, TAIL_ROWS)],
                            acc_sh.at[pl.ds(N_FULL_OUT * K, TAIL_ROWS)])

        plsc.subcore_barrier()

        def scale_rows(rows, ewb):
            @pl.loop(0, K // 16)
            def _scale_group(g):
                ewg = ewb[pl.ds(g * 16, 16)]
                for jj in range(16):
                    e = g * 16 + jj
                    w = ewg[jj]
                    for c in range(D // 16):
                        rows[e, pl.ds(c * 16, 16)] = (
                            rows[e, pl.ds(c * 16, 16)] * w)

        pltpu.sync_copy(src_hbm.at[pl.ds(base_chunk * K, K)], sb0)
        pltpu.sync_copy(dst_hbm.at[pl.ds(base_chunk * K, K)], db0)
        pltpu.sync_copy(ew_hbm.at[pl.ds(base_chunk * K, K)], ewb0)
        pltpu.async_copy(x_hbm.at[sb0.at[pl.ds(0, K // 2)]],
                         rows0.at[pl.ds(0, K // 2)], gsem0)
        pltpu.async_copy(x_hbm.at[sb0.at[pl.ds(K // 2, K // 2)]],
                         rows0.at[pl.ds(K // 2, K // 2)], hsem0)

        def halfstep(j, cur):
            nxt = 1 - cur
            nbase = (base_chunk + j + 1) * K
            pltpu.make_async_copy(
                x_hbm.at[sbs[cur].at[pl.ds(0, K // 2)]],
                rowss[cur].at[pl.ds(0, K // 2)], gsems[cur]).wait()
            pltpu.make_async_copy(
                x_hbm.at[sbs[cur].at[pl.ds(K // 2, K // 2)]],
                rowss[cur].at[pl.ds(K // 2, K // 2)], hsems[cur]).wait()
            pltpu.async_copy(src_hbm.at[pl.ds(nbase, K)], sbs[nxt], isems[nxt])
            pltpu.async_copy(dst_hbm.at[pl.ds(nbase, K)], dbs[nxt], isems[nxt])
            pltpu.async_copy(ew_hbm.at[pl.ds(nbase, K)], ewbs[nxt], isems[nxt])
            scale_rows(rowss[cur], ewbs[cur])
            pltpu.make_async_copy(src_hbm.at[pl.ds(nbase, K)], sbs[nxt],
                                  isems[nxt]).wait()
            pltpu.make_async_copy(dst_hbm.at[pl.ds(nbase, K)], dbs[nxt],
                                  isems[nxt]).wait()
            pltpu.make_async_copy(ew_hbm.at[pl.ds(nbase, K)], ewbs[nxt],
                                  isems[nxt]).wait()
            pltpu.async_copy(x_hbm.at[sbs[nxt].at[pl.ds(0, K // 2)]],
                             rowss[nxt].at[pl.ds(0, K // 2)], gsems[nxt])
            pltpu.async_copy(x_hbm.at[sbs[nxt].at[pl.ds(K // 2, K // 2)]],
                             rowss[nxt].at[pl.ds(K // 2, K // 2)], hsems[nxt])
            pltpu.sync_copy(rowss[cur], acc_sh.at[dbs[cur]], add=True)

        @pl.loop(0, n_pairs)
        def _pair(t):
            halfstep(2 * t, 0)
            halfstep(2 * t + 1, 1)

        pltpu.make_async_copy(x_hbm.at[sb0.at[pl.ds(0, K // 2)]],
                              rows0.at[pl.ds(0, K // 2)], gsem0).wait()
        pltpu.make_async_copy(x_hbm.at[sb0.at[pl.ds(K // 2, K // 2)]],
                              rows0.at[pl.ds(K // 2, K // 2)], hsem0).wait()

        plsc.subcore_barrier()

        @pl.loop(sid, N_FULL_OUT, step=NS)
        def _copy_out(t):
            row0 = t * K
            pltpu.sync_copy(acc_sh.at[pl.ds(row0, K)], rows0)
            pltpu.sync_copy(rows0, part_hbm.at[cid, pl.ds(row0, K)])

        @pl.when(sid == 0)
        def _copy_tail():
            row0 = N_FULL_OUT * K
            pltpu.sync_copy(acc_sh.at[pl.ds(row0, TAIL_ROWS)],
                            rows1.at[pl.ds(0, TAIL_ROWS)])
            pltpu.sync_copy(rows1.at[pl.ds(0, TAIL_ROWS)],
                            part_hbm.at[cid, pl.ds(row0, TAIL_ROWS)])

    return agg(x, srcs, dsts, ew)


def _combine_body(p_ref, w_ref, o_ref):
    o_ref[...] = (p_ref[0] + p_ref[1]) * w_ref[...]


def _tc_combine(part, W):
    blk = 2000
    return pl.pallas_call(
        _combine_body,
        out_shape=jax.ShapeDtypeStruct((N, D), jnp.float32),
        grid=(N // blk,),
        in_specs=[
            pl.BlockSpec((NC, blk, D), lambda i: (0, i, 0)),
            pl.BlockSpec((1, D), lambda i: (0, 0)),
        ],
        out_specs=pl.BlockSpec((blk, D), lambda i: (i, 0)),
    )(part, W.reshape(1, D))


def kernel(x, edge_index, edge_weight, W):
    dst = edge_index[0]
    src = edge_index[1]
    pad = E_PAD - E + K
    src_p = jnp.concatenate([src, jnp.zeros((pad,), jnp.int32)])
    dst_p = jnp.concatenate([dst, jnp.zeros((pad,), jnp.int32)])
    ew_p = jnp.concatenate([edge_weight, jnp.zeros((pad,), jnp.float32)])
    part = _sc_aggregate(x, src_p, dst_p, ew_p)
    return _tc_combine(part, W)


# speedup vs baseline: 1.2298x; 1.1433x over previous
import functools

import jax
import jax.numpy as jnp
from jax import lax
from jax.experimental import pallas as pl
from jax.experimental.pallas import tpu as pltpu
from jax.experimental.pallas import tpu_sc as plsc

N = 10000
D = 128
E = 320000

K = 128
NC = 2
NS = 16
NW = NC * NS
CH0 = 132
CH1 = 28
NWC = NS * (CH0 + CH1)
E_PAD = NWC * K

N_FULL_OUT = N // K
TAIL_ROWS = N - N_FULL_OUT * K


def _sc_aggregate(x, srcs, dsts, ew):
    mesh = plsc.VectorSubcoreMesh(core_axis_name="c", subcore_axis_name="s")

    @functools.partial(
        pl.kernel,
        out_type=jax.ShapeDtypeStruct((NC, N, D), jnp.float32),
        mesh=mesh,
        compiler_params=pltpu.CompilerParams(use_tc_tiling_on_sc=False),
        scratch_types=[
            pltpu.VMEM((K,), jnp.int32),
            pltpu.VMEM((K,), jnp.int32),
            pltpu.VMEM((K,), jnp.int32),
            pltpu.VMEM((K,), jnp.int32),
            pltpu.VMEM((K,), jnp.float32),
            pltpu.VMEM((K,), jnp.float32),
            pltpu.VMEM((K, D), jnp.float32),
            pltpu.VMEM((K, D), jnp.float32),
            pltpu.VMEM_SHARED((N, D), jnp.float32),
            pltpu.SemaphoreType.DMA,
            pltpu.SemaphoreType.DMA,
            pltpu.SemaphoreType.DMA,
            pltpu.SemaphoreType.DMA,
            pltpu.SemaphoreType.DMA,
            pltpu.SemaphoreType.DMA,
        ],
    )
    def agg(x_hbm, src_hbm, dst_hbm, ew_hbm, part_hbm,
            sb0, sb1, db0, db1, ewb0, ewb1, rows0, rows1, acc_sh,
            isem0, isem1, gsem0, gsem1, hsem0, hsem1):
        cid = lax.axis_index("c")
        sid = lax.axis_index("s")
        base_chunk = jnp.where(cid == 0, sid * CH0, NS * CH0 + sid * CH1)
        n_pairs = jnp.where(cid == 0, CH0 // 2, CH1 // 2)

        sbs = (sb0, sb1)
        dbs = (db0, db1)
        ewbs = (ewb0, ewb1)
        rowss = (rows0, rows1)
        isems = (isem0, isem1)
        gsems = (gsem0, gsem1)
        hsems = (hsem0, hsem1)

        zero16 = jnp.zeros((16,), jnp.float32)

        @pl.loop(0, K)
        def _zero_rows(r):
            for c in range(D // 16):
                rows0[r, pl.ds(c * 16, 16)] = zero16

        @pl.loop(sid, N_FULL_OUT, step=NS)
        def _zero_acc(t):
            pltpu.sync_copy(rows0, acc_sh.at[pl.ds(t * K, K)])

        @pl.when(sid == 0)
        def _zero_tail():
            pltpu.sync_copy(rows0.at[pl.ds(0, TAIL_ROWS)],
                            acc_sh.at[pl.ds(N_FULL_OUT * K, TAIL_ROWS)])

        plsc.subcore_barrier()

        def scale_rows(rows, ewb):
            @pl.loop(0, K // 16)
            def _scale_group(g):
                ewg = ewb[pl.ds(g * 16, 16)]
                for jj in range(16):
                    e = g * 16 + jj
                    w = ewg[jj]
                    for c in range(D // 16):
                        rows[e, pl.ds(c * 16, 16)] = (
                            rows[e, pl.ds(c * 16, 16)] * w)

        pltpu.sync_copy(src_hbm.at[pl.ds(base_chunk * K, K)], sb0)
        pltpu.sync_copy(dst_hbm.at[pl.ds(base_chunk * K, K)], db0)
        pltpu.sync_copy(ew_hbm.at[pl.ds(base_chunk * K, K)], ewb0)
        pltpu.async_copy(x_hbm.at[sb0.at[pl.ds(0, K // 2)]],
                         rows0.at[pl.ds(0, K // 2)], gsem0)
        pltpu.async_copy(x_hbm.at[sb0.at[pl.ds(K // 2, K // 2)]],
                         rows0.at[pl.ds(K // 2, K // 2)], hsem0)

        def halfstep(j, cur):
            nxt = 1 - cur
            nbase = (base_chunk + j + 1) * K
            pltpu.make_async_copy(
                x_hbm.at[sbs[cur].at[pl.ds(0, K // 2)]],
                rowss[cur].at[pl.ds(0, K // 2)], gsems[cur]).wait()
            pltpu.make_async_copy(
                x_hbm.at[sbs[cur].at[pl.ds(K // 2, K // 2)]],
                rowss[cur].at[pl.ds(K // 2, K // 2)], hsems[cur]).wait()
            pltpu.async_copy(src_hbm.at[pl.ds(nbase, K)], sbs[nxt], isems[nxt])
            pltpu.async_copy(dst_hbm.at[pl.ds(nbase, K)], dbs[nxt], isems[nxt])
            pltpu.async_copy(ew_hbm.at[pl.ds(nbase, K)], ewbs[nxt], isems[nxt])
            scale_rows(rowss[cur], ewbs[cur])
            pltpu.make_async_copy(src_hbm.at[pl.ds(nbase, K)], sbs[nxt],
                                  isems[nxt]).wait()
            pltpu.make_async_copy(dst_hbm.at[pl.ds(nbase, K)], dbs[nxt],
                                  isems[nxt]).wait()
            pltpu.make_async_copy(ew_hbm.at[pl.ds(nbase, K)], ewbs[nxt],
                                  isems[nxt]).wait()
            pltpu.async_copy(x_hbm.at[sbs[nxt].at[pl.ds(0, K // 2)]],
                             rowss[nxt].at[pl.ds(0, K // 2)], gsems[nxt])
            pltpu.async_copy(x_hbm.at[sbs[nxt].at[pl.ds(K // 2, K // 2)]],
                             rowss[nxt].at[pl.ds(K // 2, K // 2)], hsems[nxt])
            pltpu.sync_copy(rowss[cur], acc_sh.at[dbs[cur]], add=True)

        @pl.loop(0, n_pairs)
        def _pair(t):
            halfstep(2 * t, 0)
            halfstep(2 * t + 1, 1)

        pltpu.make_async_copy(x_hbm.at[sb0.at[pl.ds(0, K // 2)]],
                              rows0.at[pl.ds(0, K // 2)], gsem0).wait()
        pltpu.make_async_copy(x_hbm.at[sb0.at[pl.ds(K // 2, K // 2)]],
                              rows0.at[pl.ds(K // 2, K // 2)], hsem0).wait()

        plsc.subcore_barrier()

        @pl.loop(sid, N_FULL_OUT, step=NS)
        def _copy_out(t):
            row0 = t * K
            pltpu.sync_copy(acc_sh.at[pl.ds(row0, K)], rows0)
            pltpu.sync_copy(rows0, part_hbm.at[cid, pl.ds(row0, K)])

        @pl.when(sid == 0)
        def _copy_tail():
            row0 = N_FULL_OUT * K
            pltpu.sync_copy(acc_sh.at[pl.ds(row0, TAIL_ROWS)],
                            rows1.at[pl.ds(0, TAIL_ROWS)])
            pltpu.sync_copy(rows1.at[pl.ds(0, TAIL_ROWS)],
                            part_hbm.at[cid, pl.ds(row0, TAIL_ROWS)])

    return agg(x, srcs, dsts, ew)


def _combine_body(p_ref, w_ref, o_ref):
    o_ref[...] = (p_ref[0] + p_ref[1]) * w_ref[...]


def _tc_combine(part, W):
    blk = 2000
    return pl.pallas_call(
        _combine_body,
        out_shape=jax.ShapeDtypeStruct((N, D), jnp.float32),
        grid=(N // blk,),
        in_specs=[
            pl.BlockSpec((NC, blk, D), lambda i: (0, i, 0)),
            pl.BlockSpec((1, D), lambda i: (0, 0)),
        ],
        out_specs=pl.BlockSpec((blk, D), lambda i: (i, 0)),
    )(part, W.reshape(1, D))


def kernel(x, edge_index, edge_weight, W):
    dst = edge_index[0]
    src = edge_index[1]
    pad = E_PAD - E + K
    src_p = jnp.concatenate([src, jnp.zeros((pad,), jnp.int32)])
    dst_p = jnp.concatenate([dst, jnp.zeros((pad,), jnp.int32)])
    ew_p = jnp.concatenate([edge_weight, jnp.zeros((pad,), jnp.float32)])
    part = _sc_aggregate(x, src_p, dst_p, ew_p)
    return _tc_combine(part, W)


# speedup vs baseline: 1.4207x; 1.1552x over previous
import functools

import jax
import jax.numpy as jnp
from jax import lax
from jax.experimental import pallas as pl
from jax.experimental.pallas import tpu as pltpu
from jax.experimental.pallas import tpu_sc as plsc

N = 10000
D = 128
E = 320000

K = 128
NC = 2
NS = 16
NW = NC * NS
CH0 = 144
CH1 = 16
NWC = NS * (CH0 + CH1)
E_PAD = NWC * K

N_FULL_OUT = N // K
TAIL_ROWS = N - N_FULL_OUT * K


def _sc_aggregate(x, srcs, dsts, ew):
    mesh = plsc.VectorSubcoreMesh(core_axis_name="c", subcore_axis_name="s")

    @functools.partial(
        pl.kernel,
        out_type=jax.ShapeDtypeStruct((NC, N, D), jnp.float32),
        mesh=mesh,
        compiler_params=pltpu.CompilerParams(use_tc_tiling_on_sc=False),
        scratch_types=[
            pltpu.VMEM((K,), jnp.int32),
            pltpu.VMEM((K,), jnp.int32),
            pltpu.VMEM((K,), jnp.int32),
            pltpu.VMEM((K,), jnp.int32),
            pltpu.VMEM((K,), jnp.float32),
            pltpu.VMEM((K,), jnp.float32),
            pltpu.VMEM((K, D), jnp.float32),
            pltpu.VMEM((K, D), jnp.float32),
            pltpu.VMEM_SHARED((N, D), jnp.float32),
            pltpu.SemaphoreType.DMA,
            pltpu.SemaphoreType.DMA,
            pltpu.SemaphoreType.DMA,
            pltpu.SemaphoreType.DMA,
            pltpu.SemaphoreType.DMA,
            pltpu.SemaphoreType.DMA,
        ],
    )
    def agg(x_hbm, src_hbm, dst_hbm, ew_hbm, part_hbm,
            sb0, sb1, db0, db1, ewb0, ewb1, rows0, rows1, acc_sh,
            isem0, isem1, gsem0, gsem1, hsem0, hsem1):
        cid = lax.axis_index("c")
        sid = lax.axis_index("s")
        base_chunk = jnp.where(cid == 0, sid * CH0, NS * CH0 + sid * CH1)
        n_pairs = jnp.where(cid == 0, CH0 // 2, CH1 // 2)

        sbs = (sb0, sb1)
        dbs = (db0, db1)
        ewbs = (ewb0, ewb1)
        rowss = (rows0, rows1)
        isems = (isem0, isem1)
        gsems = (gsem0, gsem1)
        hsems = (hsem0, hsem1)

        zero16 = jnp.zeros((16,), jnp.float32)

        @pl.loop(0, K)
        def _zero_rows(r):
            for c in range(D // 16):
                rows0[r, pl.ds(c * 16, 16)] = zero16

        @pl.loop(sid, N_FULL_OUT, step=NS)
        def _zero_acc(t):
            pltpu.sync_copy(rows0, acc_sh.at[pl.ds(t * K, K)])

        @pl.when(sid == 0)
        def _zero_tail():
            pltpu.sync_copy(rows0.at[pl.ds(0, TAIL_ROWS)],
                            acc_sh.at[pl.ds(N_FULL_OUT * K, TAIL_ROWS)])

        plsc.subcore_barrier()

        def scale_rows(rows, ewb):
            @pl.loop(0, K // 16)
            def _scale_group(g):
                ewg = ewb[pl.ds(g * 16, 16)]
                for jj in range(16):
                    e = g * 16 + jj
                    w = ewg[jj]
                    for c in range(D // 16):
                        rows[e, pl.ds(c * 16, 16)] = (
                            rows[e, pl.ds(c * 16, 16)] * w)

        pltpu.sync_copy(src_hbm.at[pl.ds(base_chunk * K, K)], sb0)
        pltpu.sync_copy(dst_hbm.at[pl.ds(base_chunk * K, K)], db0)
        pltpu.sync_copy(ew_hbm.at[pl.ds(base_chunk * K, K)], ewb0)
        pltpu.async_copy(x_hbm.at[sb0.at[pl.ds(0, K // 2)]],
                         rows0.at[pl.ds(0, K // 2)], gsem0)
        pltpu.async_copy(x_hbm.at[sb0.at[pl.ds(K // 2, K // 2)]],
                         rows0.at[pl.ds(K // 2, K // 2)], hsem0)

        def halfstep(j, cur):
            nxt = 1 - cur
            nbase = (base_chunk + j + 1) * K
            pltpu.make_async_copy(
                x_hbm.at[sbs[cur].at[pl.ds(0, K // 2)]],
                rowss[cur].at[pl.ds(0, K // 2)], gsems[cur]).wait()
            pltpu.make_async_copy(
                x_hbm.at[sbs[cur].at[pl.ds(K // 2, K // 2)]],
                rowss[cur].at[pl.ds(K // 2, K // 2)], hsems[cur]).wait()
            pltpu.async_copy(src_hbm.at[pl.ds(nbase, K)], sbs[nxt], isems[nxt])
            pltpu.async_copy(dst_hbm.at[pl.ds(nbase, K)], dbs[nxt], isems[nxt])
            pltpu.async_copy(ew_hbm.at[pl.ds(nbase, K)], ewbs[nxt], isems[nxt])
            scale_rows(rowss[cur], ewbs[cur])
            pltpu.make_async_copy(src_hbm.at[pl.ds(nbase, K)], sbs[nxt],
                                  isems[nxt]).wait()
            pltpu.make_async_copy(dst_hbm.at[pl.ds(nbase, K)], dbs[nxt],
                                  isems[nxt]).wait()
            pltpu.make_async_copy(ew_hbm.at[pl.ds(nbase, K)], ewbs[nxt],
                                  isems[nxt]).wait()
            pltpu.async_copy(x_hbm.at[sbs[nxt].at[pl.ds(0, K // 2)]],
                             rowss[nxt].at[pl.ds(0, K // 2)], gsems[nxt])
            pltpu.async_copy(x_hbm.at[sbs[nxt].at[pl.ds(K // 2, K // 2)]],
                             rowss[nxt].at[pl.ds(K // 2, K // 2)], hsems[nxt])
            pltpu.sync_copy(rowss[cur], acc_sh.at[dbs[cur]], add=True)

        @pl.loop(0, n_pairs)
        def _pair(t):
            halfstep(2 * t, 0)
            halfstep(2 * t + 1, 1)

        pltpu.make_async_copy(x_hbm.at[sb0.at[pl.ds(0, K // 2)]],
                              rows0.at[pl.ds(0, K // 2)], gsem0).wait()
        pltpu.make_async_copy(x_hbm.at[sb0.at[pl.ds(K // 2, K // 2)]],
                              rows0.at[pl.ds(K // 2, K // 2)], hsem0).wait()

        plsc.subcore_barrier()

        @pl.loop(sid, N_FULL_OUT, step=NS)
        def _copy_out(t):
            row0 = t * K
            pltpu.sync_copy(acc_sh.at[pl.ds(row0, K)], rows0)
            pltpu.sync_copy(rows0, part_hbm.at[cid, pl.ds(row0, K)])

        @pl.when(sid == 0)
        def _copy_tail():
            row0 = N_FULL_OUT * K
            pltpu.sync_copy(acc_sh.at[pl.ds(row0, TAIL_ROWS)],
                            rows1.at[pl.ds(0, TAIL_ROWS)])
            pltpu.sync_copy(rows1.at[pl.ds(0, TAIL_ROWS)],
                            part_hbm.at[cid, pl.ds(row0, TAIL_ROWS)])

    return agg(x, srcs, dsts, ew)


def _combine_body(p_ref, w_ref, o_ref):
    o_ref[...] = (p_ref[0] + p_ref[1]) * w_ref[...]


def _tc_combine(part, W):
    blk = 2000
    return pl.pallas_call(
        _combine_body,
        out_shape=jax.ShapeDtypeStruct((N, D), jnp.float32),
        grid=(N // blk,),
        in_specs=[
            pl.BlockSpec((NC, blk, D), lambda i: (0, i, 0)),
            pl.BlockSpec((1, D), lambda i: (0, 0)),
        ],
        out_specs=pl.BlockSpec((blk, D), lambda i: (i, 0)),
    )(part, W.reshape(1, D))


def kernel(x, edge_index, edge_weight, W):
    dst = edge_index[0]
    src = edge_index[1]
    pad = E_PAD - E + K
    src_p = jnp.concatenate([src, jnp.zeros((pad,), jnp.int32)])
    dst_p = jnp.concatenate([dst, jnp.zeros((pad,), jnp.int32)])
    ew_p = jnp.concatenate([edge_weight, jnp.zeros((pad,), jnp.float32)])
    part = _sc_aggregate(x, src_p, dst_p, ew_p)
    return _tc_combine(part, W)
